# baseline (speedup 1.0000x reference)
import functools

import jax
import jax.numpy as jnp
from jax import lax
from jax.experimental import pallas as pl
from jax.experimental.pallas import tpu as pltpu
from jax.experimental.pallas import tpu_sc as plsc

N = 320000
F = 128
NC, NS = 2, 16
NW = NC * NS
RW = N // NW
CH = 400
NCH = RW // CH
U = 16

_mesh = plsc.VectorSubcoreMesh(core_axis_name="c", subcore_axis_name="s")


@functools.partial(
    pl.kernel,
    mesh=_mesh,
    out_type=jax.ShapeDtypeStruct((N,), jnp.float32),
    scratch_types=[
        pltpu.VMEM((CH, F), jnp.float32),
        pltpu.VMEM((CH, F), jnp.float32),
        pltpu.VMEM((CH,), jnp.float32),
    ],
)
def _sc_dots(x1_hbm, x2_hbm, out_hbm, x1v, x2v, dv):
    wid = lax.axis_index("s") * NC + lax.axis_index("c")
    base = wid * RW

    def chunk_body(c, carry):
        off = base + c * CH
        pltpu.sync_copy(x1_hbm.at[pl.ds(off, CH), :], x1v)
        pltpu.sync_copy(x2_hbm.at[pl.ds(off, CH), :], x2v)

        lanes = lax.iota(jnp.int32, 16)

        def row_body(r, carry2):
            vec = jnp.zeros((16,), jnp.float32)
            for u in range(U):
                rr = r * U + u
                acc = x1v[rr, pl.ds(0, 16)] * x2v[rr, pl.ds(0, 16)]
                for j in range(1, F // 16):
                    acc = acc + x1v[rr, pl.ds(j * 16, 16)] * x2v[rr, pl.ds(j * 16, 16)]
                for s in (8, 4, 2, 1):
                    acc = acc + acc.at[lanes ^ s].get(mode="promise_in_bounds")
                vec = jnp.where(lanes == u, acc, vec)
            dv[pl.ds(r * U, U)] = vec
            return carry2

        lax.fori_loop(0, CH // U, row_body, 0)
        pltpu.sync_copy(dv, out_hbm.at[pl.ds(off, CH)])
        return carry

    lax.fori_loop(0, NCH, chunk_body, 0)


def _softmax_body(d_ref, out_ref):
    d = d_ref[...]
    m = jnp.max(d)
    e = jnp.exp(d - m)
    out_ref[...] = e / jnp.sum(e)


def kernel(node1, node2):
    x1 = node1.reshape(N, F)
    x2 = node2.reshape(N, F)
    dots = _sc_dots(x1, x2)
    res = pl.pallas_call(
        _softmax_body,
        out_shape=jax.ShapeDtypeStruct((N // F, F), jnp.float32),
    )(dots.reshape(N // F, F))
    return res.reshape(N, 1)


# speedup vs baseline: 3.4422x; 3.4422x over previous
import functools

import jax
import jax.numpy as jnp
from jax import lax
from jax.experimental import pallas as pl
from jax.experimental.pallas import tpu as pltpu
from jax.experimental.pallas import tpu_sc as plsc

N = 320000
F = 128

NC, NS = 2, 16
NW = NC * NS
CH = 400
S = 64000
RW = S // NW
NCH = RW // CH
U = 16

T = N - S
TGRID = 20
TROWS = T // TGRID
TG = TROWS // F
SOFF = S // TROWS

_mesh = plsc.VectorSubcoreMesh(core_axis_name="c", subcore_axis_name="s")


@functools.partial(
    pl.kernel,
    mesh=_mesh,
    out_type=jax.ShapeDtypeStruct((S,), jnp.float32),
    scratch_types=[
        pltpu.VMEM((CH, F), jnp.float32),
        pltpu.VMEM((CH, F), jnp.float32),
        pltpu.VMEM((CH,), jnp.float32),
    ],
)
def _sc_dots(x1_hbm, x2_hbm, out_hbm, x1v, x2v, dv):
    wid = lax.axis_index("s") * NC + lax.axis_index("c")
    base = wid * RW

    def chunk_body(c, carry):
        off = base + c * CH
        pltpu.sync_copy(x1_hbm.at[pl.ds(off, CH), :], x1v)
        pltpu.sync_copy(x2_hbm.at[pl.ds(off, CH), :], x2v)

        lanes = lax.iota(jnp.int32, 16)

        def row_body(r, carry2):
            vec = jnp.zeros((16,), jnp.float32)
            for u in range(U):
                rr = r * U + u
                acc = x1v[rr, pl.ds(0, 16)] * x2v[rr, pl.ds(0, 16)]
                for j in range(1, F // 16):
                    acc = acc + x1v[rr, pl.ds(j * 16, 16)] * x2v[rr, pl.ds(j * 16, 16)]
                for s in (8, 4, 2, 1):
                    acc = acc + acc.at[lanes ^ s].get(mode="promise_in_bounds")
                vec = jnp.where(lanes == u, acc, vec)
            dv[pl.ds(r * U, U)] = vec
            return carry2

        lax.fori_loop(0, CH // U, row_body, 0)
        pltpu.sync_copy(dv, out_hbm.at[pl.ds(off, CH)])
        return carry

    lax.fori_loop(0, NCH, chunk_body, 0)


def _tc_dots_body(x1_ref, x2_ref, out_ref):
    prod = x1_ref[...] * x2_ref[...]
    out_ref[0, :, :] = jnp.sum(prod.reshape(TG, F, F), axis=2)


def _combine_body(sc_ref, tc_ref, out_ref):
    a = sc_ref[...]
    b = tc_ref[...]
    m = jnp.maximum(jnp.max(a), jnp.max(b))
    ea = jnp.exp(a - m)
    eb = jnp.exp(b - m)
    s = jnp.sum(ea) + jnp.sum(eb)
    out_ref[0:S // F, :] = ea / s
    out_ref[S // F:, :] = eb / s


def kernel(node1, node2):
    x1 = node1.reshape(N, F)
    x2 = node2.reshape(N, F)

    dots_sc = _sc_dots(x1, x2)

    dots_tc = pl.pallas_call(
        _tc_dots_body,
        grid=(TGRID,),
        in_specs=[
            pl.BlockSpec((TROWS, F), lambda i: (i + SOFF, 0)),
            pl.BlockSpec((TROWS, F), lambda i: (i + SOFF, 0)),
        ],
        out_specs=pl.BlockSpec((1, TG, F), lambda i: (i, 0, 0)),
        out_shape=jax.ShapeDtypeStruct((TGRID, TG, F), jnp.float32),
    )(x1, x2)

    res = pl.pallas_call(
        _combine_body,
        out_shape=jax.ShapeDtypeStruct((N // F, F), jnp.float32),
    )(dots_sc.reshape(S // F, F), dots_tc.reshape(T // F, F))
    return res.reshape(N, 1)
